# baseline (speedup 1.0000x reference)
import functools

import jax
import jax.numpy as jnp
from jax import lax
from jax.experimental import pallas as pl
from jax.experimental.pallas import tpu as pltpu
from jax.experimental.pallas import tpu_sc as plsc

DIM = 64
LANES = 16
CHUNK = 128
NSLOT = 4


def _bcast_lane(v, j):
    return lax.gather(
        v,
        jnp.full((LANES, 1), j, jnp.int32),
        lax.GatherDimensionNumbers(
            offset_dims=(), collapsed_slice_dims=(0,), start_index_map=(0,)),
        (1,),
        mode=lax.GatherScatterMode.PROMISE_IN_BOUNDS)


@functools.cache
def _build_sc_kernel(n_tokens: int):
    info = plsc.get_sparse_core_info()
    n_workers = info.num_cores * info.num_subcores
    per_worker = n_tokens // n_workers
    n_chunks = per_worker // CHUNK
    n_iters = n_chunks // NSLOT
    assert per_worker * n_workers == n_tokens
    assert n_iters * NSLOT == n_chunks

    mesh = plsc.VectorSubcoreMesh(core_axis_name="c", subcore_axis_name="s")

    @functools.partial(
        pl.kernel,
        mesh=mesh,
        out_type=jax.ShapeDtypeStruct((n_tokens, DIM), jnp.float32),
        compiler_params=pltpu.CompilerParams(use_tc_tiling_on_sc=False),
        scratch_types=[
            pltpu.VMEM((per_worker,), jnp.int32),
            pltpu.VMEM((per_worker,), jnp.float32),
            pltpu.VMEM((NSLOT, CHUNK, DIM), jnp.float32),
            pltpu.VMEM((NSLOT, CHUNK, DIM), jnp.float32),
            pltpu.SemaphoreType.DMA((NSLOT,)),
            pltpu.SemaphoreType.DMA((NSLOT,)),
        ],
    )
    def sc_combine(x_hbm, ctx_hbm, base_hbm, super_hbm, out_hbm,
                   idx_all, ctx_all, s_v, o_v, gsem, osem):
        wid = lax.axis_index("s") * info.num_cores + lax.axis_index("c")
        w_base = wid * per_worker

        pltpu.sync_copy(x_hbm.at[pl.ds(w_base, per_worker)], idx_all)
        pltpu.sync_copy(ctx_hbm.at[pl.ds(w_base, per_worker)], ctx_all)

        def fire_gathers(c, k):
            idx_slice = idx_all.at[pl.ds(c * CHUNK, CHUNK)]
            pltpu.async_copy(base_hbm.at[idx_slice], o_v.at[k], gsem.at[k])
            pltpu.async_copy(super_hbm.at[idx_slice], s_v.at[k], gsem.at[k])

        def drain_gathers(c, k):
            idx_slice = idx_all.at[pl.ds(c * CHUNK, CHUNK)]
            pltpu.make_async_copy(
                base_hbm.at[idx_slice], o_v.at[k], gsem.at[k]).wait()
            pltpu.make_async_copy(
                super_hbm.at[idx_slice], s_v.at[k], gsem.at[k]).wait()

        def fire_scatter(c, k):
            pltpu.async_copy(
                o_v.at[k], out_hbm.at[pl.ds(w_base + c * CHUNK, CHUNK)],
                osem.at[k])

        def drain_scatter(c, k):
            pltpu.make_async_copy(
                o_v.at[k], out_hbm.at[pl.ds(w_base + c * CHUNK, CHUNK)],
                osem.at[k]).wait()

        def compute(c, k):
            s_ref = s_v.at[k]
            o_ref = o_v.at[k]
            goff = c * CHUNK

            def group(tg, carry):
                t0 = tg * LANES
                cv16 = ctx_all[pl.ds(goff + t0, LANES)]
                for j in range(LANES):
                    cb = _bcast_lane(cv16, j)
                    t = t0 + j
                    for d in range(DIM // LANES):
                        sl = pl.ds(d * LANES, LANES)
                        plsc.addupdate(o_ref.at[t, sl], cb * s_ref[t, sl])
                return carry

            lax.fori_loop(0, CHUNK // LANES, group, 0)

        for k in range(NSLOT - 1):
            fire_gathers(k, k)

        def iter_body(q, carry):
            c0 = q * NSLOT
            for k in range(NSLOT):
                c = c0 + k
                drain_gathers(c, k)
                compute(c, k)
                fire_scatter(c, k)
                kn = (k + NSLOT - 1) % NSLOT
                if k == 0:
                    @pl.when(q > 0)
                    def _():
                        drain_scatter(c - 1, kn)
                    fire_gathers(c + NSLOT - 1, kn)
                else:
                    drain_scatter(c - 1, kn)
                    @pl.when(q < n_iters - 1)
                    def _():
                        fire_gathers(c + NSLOT - 1, kn)
            return carry

        lax.fori_loop(0, n_iters, iter_body, 0)
        drain_scatter(n_chunks - 1, (n_chunks - 1) % NSLOT)

    return sc_combine


def _t2_body(in_ref, out_ref):
    out_ref[...] = in_ref[...].T


def _tc_transpose2(a, col_block=512):
    r, c = a.shape
    return pl.pallas_call(
        _t2_body,
        grid=(pl.cdiv(c, col_block),),
        in_specs=[pl.BlockSpec((r, col_block), lambda j: (0, j))],
        out_specs=pl.BlockSpec((col_block, r), lambda j: (j, 0)),
        out_shape=jax.ShapeDtypeStruct((c, r), a.dtype),
    )(a)


def _t3_body(in_ref, out_ref):
    out_ref[...] = jnp.transpose(in_ref[...], (0, 2, 1))


def _tc_transpose3(a, col_block=512):
    l, c, d = a.shape
    return pl.pallas_call(
        _t3_body,
        grid=(l, pl.cdiv(c, col_block)),
        in_specs=[pl.BlockSpec((1, col_block, d), lambda i, j: (i, j, 0))],
        out_specs=pl.BlockSpec((1, d, col_block), lambda i, j: (i, 0, j)),
        out_shape=jax.ShapeDtypeStruct((l, d, c), a.dtype),
    )(a)


def kernel(x, context_vector, base_table, super_table):
    b, l = x.shape
    n_tokens = b * l
    base_rm = _tc_transpose2(base_table.T)
    super_rm = _tc_transpose2(super_table.T)
    xt = jnp.swapaxes(x, 0, 1).reshape(n_tokens).astype(jnp.int32)
    ct = jnp.swapaxes(context_vector, 0, 1).reshape(n_tokens)
    sc = _build_sc_kernel(n_tokens)
    out = sc(xt, ct, base_rm, super_rm)
    out_t = _tc_transpose3(out.reshape(l, b, DIM))
    return jnp.transpose(out_t, (2, 0, 1))


# speedup vs baseline: 2.3069x; 2.3069x over previous
import functools

import jax
import jax.numpy as jnp
from jax import lax
from jax.experimental import pallas as pl
from jax.experimental.pallas import tpu as pltpu
from jax.experimental.pallas import tpu_sc as plsc

DIM = 64
LANES = 16
CHUNK = 128
NSLOT = 4


def _bcast_lane(v, j):
    return lax.gather(
        v,
        jnp.full((LANES, 1), j, jnp.int32),
        lax.GatherDimensionNumbers(
            offset_dims=(), collapsed_slice_dims=(0,), start_index_map=(0,)),
        (1,),
        mode=lax.GatherScatterMode.PROMISE_IN_BOUNDS)


@functools.cache
def _build_sc_kernel(n_tokens: int):
    info = plsc.get_sparse_core_info()
    n_workers = info.num_cores * info.num_subcores
    per_worker = n_tokens // n_workers
    n_chunks = per_worker // CHUNK
    n_iters = n_chunks // NSLOT
    assert per_worker * n_workers == n_tokens
    assert n_iters * NSLOT == n_chunks

    mesh = plsc.VectorSubcoreMesh(core_axis_name="c", subcore_axis_name="s")

    @functools.partial(
        pl.kernel,
        mesh=mesh,
        out_type=jax.ShapeDtypeStruct((n_tokens, DIM), jnp.float32),
        compiler_params=pltpu.CompilerParams(use_tc_tiling_on_sc=False),
        scratch_types=[
            pltpu.VMEM((per_worker,), jnp.int32),
            pltpu.VMEM((per_worker,), jnp.float32),
            pltpu.VMEM((NSLOT, CHUNK, DIM), jnp.float32),
            pltpu.VMEM((NSLOT, CHUNK, DIM), jnp.float32),
            pltpu.SemaphoreType.DMA((NSLOT,)),
            pltpu.SemaphoreType.DMA((NSLOT,)),
        ],
    )
    def sc_combine(x_hbm, ctx_hbm, base_hbm, super_hbm, out_hbm,
                   idx_all, ctx_all, s_v, o_v, gsem, osem):
        wid = lax.axis_index("s") * info.num_cores + lax.axis_index("c")
        w_base = wid * per_worker

        pltpu.sync_copy(x_hbm.at[pl.ds(w_base, per_worker)], idx_all)
        pltpu.sync_copy(ctx_hbm.at[pl.ds(w_base, per_worker)], ctx_all)

        def fire_gathers(c, k):
            idx_slice = idx_all.at[pl.ds(c * CHUNK, CHUNK)]
            pltpu.async_copy(base_hbm.at[idx_slice], o_v.at[k], gsem.at[k])
            pltpu.async_copy(super_hbm.at[idx_slice], s_v.at[k], gsem.at[k])

        def drain_gathers(c, k):
            idx_slice = idx_all.at[pl.ds(c * CHUNK, CHUNK)]
            pltpu.make_async_copy(
                base_hbm.at[idx_slice], o_v.at[k], gsem.at[k]).wait()
            pltpu.make_async_copy(
                super_hbm.at[idx_slice], s_v.at[k], gsem.at[k]).wait()

        def fire_scatter(c, k):
            pltpu.async_copy(
                o_v.at[k], out_hbm.at[pl.ds(w_base + c * CHUNK, CHUNK)],
                osem.at[k])

        def drain_scatter(c, k):
            pltpu.make_async_copy(
                o_v.at[k], out_hbm.at[pl.ds(w_base + c * CHUNK, CHUNK)],
                osem.at[k]).wait()

        def compute(c, k):
            s_ref = s_v.at[k]
            o_ref = o_v.at[k]
            goff = c * CHUNK

            def group(tg, carry):
                t0 = tg * LANES
                cv16 = ctx_all[pl.ds(goff + t0, LANES)]
                for j in range(LANES):
                    cb = _bcast_lane(cv16, j)
                    t = t0 + j
                    for d in range(DIM // LANES):
                        sl = pl.ds(d * LANES, LANES)
                        plsc.addupdate(o_ref.at[t, sl], cb * s_ref[t, sl])
                return carry

            lax.fori_loop(0, CHUNK // LANES, group, 0)

        for k in range(NSLOT - 1):
            fire_gathers(k, k)

        def iter_body(q, carry):
            c0 = q * NSLOT
            for k in range(NSLOT):
                c = c0 + k
                drain_gathers(c, k)
                compute(c, k)
                fire_scatter(c, k)
                kn = (k + NSLOT - 1) % NSLOT
                if k == 0:
                    @pl.when(q > 0)
                    def _():
                        drain_scatter(c - 1, kn)
                    fire_gathers(c + NSLOT - 1, kn)
                else:
                    drain_scatter(c - 1, kn)
                    @pl.when(q < n_iters - 1)
                    def _():
                        fire_gathers(c + NSLOT - 1, kn)
            return carry

        lax.fori_loop(0, n_iters, iter_body, 0)
        drain_scatter(n_chunks - 1, (n_chunks - 1) % NSLOT)

    return sc_combine


def _t2_body(in_ref, out_ref):
    out_ref[...] = in_ref[...].T


def _tc_transpose2(a, col_block=8192):
    r, c = a.shape
    return pl.pallas_call(
        _t2_body,
        grid=(pl.cdiv(c, col_block),),
        in_specs=[pl.BlockSpec((r, col_block), lambda j: (0, j))],
        out_specs=pl.BlockSpec((col_block, r), lambda j: (j, 0)),
        out_shape=jax.ShapeDtypeStruct((c, r), a.dtype),
    )(a)


def _t3_body(in_ref, out_ref):
    out_ref[...] = jnp.transpose(in_ref[...], (0, 2, 1))


def _tc_transpose3(a, l_block=4):
    l, c, d = a.shape
    return pl.pallas_call(
        _t3_body,
        grid=(l // l_block,),
        in_specs=[pl.BlockSpec((l_block, c, d), lambda i: (i, 0, 0))],
        out_specs=pl.BlockSpec((l_block, d, c), lambda i: (i, 0, 0)),
        out_shape=jax.ShapeDtypeStruct((l, d, c), a.dtype),
    )(a)


def kernel(x, context_vector, base_table, super_table):
    b, l = x.shape
    n_tokens = b * l
    base_rm = _tc_transpose2(base_table.T)
    super_rm = _tc_transpose2(super_table.T)
    xt = jnp.swapaxes(x, 0, 1).reshape(n_tokens).astype(jnp.int32)
    ct = jnp.swapaxes(context_vector, 0, 1).reshape(n_tokens)
    sc = _build_sc_kernel(n_tokens)
    out = sc(xt, ct, base_rm, super_rm)
    out_t = _tc_transpose3(out.reshape(l, b, DIM))
    return jnp.transpose(out_t, (2, 0, 1))


# speedup vs baseline: 4.8216x; 2.0901x over previous
import functools

import jax
import jax.numpy as jnp
from jax import lax
from jax.experimental import pallas as pl
from jax.experimental.pallas import tpu as pltpu
from jax.experimental.pallas import tpu_sc as plsc

DIM = 64
ROW = 128
LANES = 16
CHUNK = 64
NSLOT = 4


def _bcast_lane(v, j):
    return lax.gather(
        v,
        jnp.full((LANES, 1), j, jnp.int32),
        lax.GatherDimensionNumbers(
            offset_dims=(), collapsed_slice_dims=(0,), start_index_map=(0,)),
        (1,),
        mode=lax.GatherScatterMode.PROMISE_IN_BOUNDS)


def _interleave_body(b_ref, s_ref, out_ref):
    out_ref[...] = jnp.concatenate([b_ref[...].T, s_ref[...].T], axis=1)


def _tc_interleave_tables(base_t, super_t, col_block=8192):
    d, v = base_t.shape
    return pl.pallas_call(
        _interleave_body,
        grid=(pl.cdiv(v, col_block),),
        in_specs=[
            pl.BlockSpec((d, col_block), lambda j: (0, j)),
            pl.BlockSpec((d, col_block), lambda j: (0, j)),
        ],
        out_specs=pl.BlockSpec((col_block, ROW), lambda j: (j, 0)),
        out_shape=jax.ShapeDtypeStruct((v, ROW), jnp.float32),
    )(base_t, super_t)


def _out_body(in_ref, out_ref):
    out_ref[...] = jnp.transpose(in_ref[:, :, 0:DIM], (0, 2, 1))


def _tc_out_transpose(a, l_block=4):
    l, b, _ = a.shape
    return pl.pallas_call(
        _out_body,
        grid=(l // l_block,),
        in_specs=[pl.BlockSpec((l_block, b, ROW), lambda i: (i, 0, 0))],
        out_specs=pl.BlockSpec((l_block, DIM, b), lambda i: (i, 0, 0)),
        out_shape=jax.ShapeDtypeStruct((l, DIM, b), jnp.float32),
    )(a)


@functools.cache
def _build_sc_kernel(n_tokens: int):
    info = plsc.get_sparse_core_info()
    n_workers = info.num_cores * info.num_subcores
    per_worker = n_tokens // n_workers
    n_chunks = per_worker // CHUNK
    n_iters = n_chunks // NSLOT
    assert per_worker * n_workers == n_tokens
    assert n_iters * NSLOT == n_chunks

    mesh = plsc.VectorSubcoreMesh(core_axis_name="c", subcore_axis_name="s")

    @functools.partial(
        pl.kernel,
        mesh=mesh,
        out_type=jax.ShapeDtypeStruct((n_tokens, ROW), jnp.float32),
        compiler_params=pltpu.CompilerParams(use_tc_tiling_on_sc=False),
        scratch_types=[
            pltpu.VMEM((per_worker,), jnp.int32),
            pltpu.VMEM((per_worker,), jnp.float32),
            pltpu.VMEM((NSLOT, CHUNK, ROW), jnp.float32),
            pltpu.VMEM((NSLOT, CHUNK, ROW), jnp.float32),
            pltpu.SemaphoreType.DMA((NSLOT,)),
            pltpu.SemaphoreType.DMA((NSLOT,)),
        ],
    )
    def sc_combine(x_hbm, ctx_hbm, tab_hbm, out_hbm,
                   idx_all, ctx_all, g_v, o_v, gsem, osem):
        wid = lax.axis_index("s") * info.num_cores + lax.axis_index("c")
        w_base = wid * per_worker

        pltpu.sync_copy(x_hbm.at[pl.ds(w_base, per_worker)], idx_all)
        pltpu.sync_copy(ctx_hbm.at[pl.ds(w_base, per_worker)], ctx_all)

        def fire_gather(c, k):
            idx_slice = idx_all.at[pl.ds(c * CHUNK, CHUNK)]
            pltpu.async_copy(tab_hbm.at[idx_slice], g_v.at[k], gsem.at[k])

        def drain_gather(c, k):
            idx_slice = idx_all.at[pl.ds(c * CHUNK, CHUNK)]
            pltpu.make_async_copy(
                tab_hbm.at[idx_slice], g_v.at[k], gsem.at[k]).wait()

        def fire_scatter(c, k):
            pltpu.async_copy(
                o_v.at[k], out_hbm.at[pl.ds(w_base + c * CHUNK, CHUNK)],
                osem.at[k])

        def drain_scatter(c, k):
            pltpu.make_async_copy(
                o_v.at[k], out_hbm.at[pl.ds(w_base + c * CHUNK, CHUNK)],
                osem.at[k]).wait()

        def compute(c, k):
            g_ref = g_v.at[k]
            o_ref = o_v.at[k]
            goff = c * CHUNK

            def group(tg, carry):
                t0 = tg * LANES
                cv16 = ctx_all[pl.ds(goff + t0, LANES)]
                for j in range(LANES):
                    cb = _bcast_lane(cv16, j)
                    t = t0 + j
                    for d in range(DIM // LANES):
                        sl = pl.ds(d * LANES, LANES)
                        sh = pl.ds(DIM + d * LANES, LANES)
                        o_ref[t, sl] = g_ref[t, sl] + cb * g_ref[t, sh]
                return carry

            lax.fori_loop(0, CHUNK // LANES, group, 0)

        for k in range(NSLOT - 1):
            fire_gather(k, k)

        def iter_body(q, carry):
            c0 = q * NSLOT
            for k in range(NSLOT):
                c = c0 + k
                drain_gather(c, k)
                @pl.when(q > 0)
                def _():
                    drain_scatter(c - NSLOT, k)
                compute(c, k)
                fire_scatter(c, k)
                kn = (k + NSLOT - 1) % NSLOT
                if k == 0:
                    fire_gather(c + NSLOT - 1, kn)
                else:
                    @pl.when(q < n_iters - 1)
                    def _():
                        fire_gather(c + NSLOT - 1, kn)
            return carry

        lax.fori_loop(0, n_iters, iter_body, 0)
        for k in range(NSLOT):
            drain_scatter(n_chunks - NSLOT + k, (n_chunks - NSLOT + k) % NSLOT)

    return sc_combine


def kernel(x, context_vector, base_table, super_table):
    b, l = x.shape
    n_tokens = b * l
    tab = _tc_interleave_tables(base_table.T, super_table.T)
    xt = jnp.swapaxes(x, 0, 1).reshape(n_tokens).astype(jnp.int32)
    ct = jnp.swapaxes(context_vector, 0, 1).reshape(n_tokens)
    sc = _build_sc_kernel(n_tokens)
    out = sc(xt, ct, tab)
    out_t = _tc_out_transpose(out.reshape(l, b, ROW))
    return jnp.transpose(out_t, (2, 0, 1))


# speedup vs baseline: 5.4501x; 1.1304x over previous
import functools

import jax
import jax.numpy as jnp
from jax import lax
from jax.experimental import pallas as pl
from jax.experimental.pallas import tpu as pltpu
from jax.experimental.pallas import tpu_sc as plsc

DIM = 64
ROW = 128
LANES = 16
CHUNK = 64
NSLOT = 4


def _bcast_lane(v, j):
    return lax.gather(
        v,
        jnp.full((LANES, 1), j, jnp.int32),
        lax.GatherDimensionNumbers(
            offset_dims=(), collapsed_slice_dims=(0,), start_index_map=(0,)),
        (1,),
        mode=lax.GatherScatterMode.PROMISE_IN_BOUNDS)


def _interleave_body(b_ref, s_ref, out_ref):
    out_ref[...] = jnp.concatenate([b_ref[...], s_ref[...]], axis=0).T


def _tc_interleave_tables(base_t, super_t, col_block=8192):
    d, v = base_t.shape
    return pl.pallas_call(
        _interleave_body,
        grid=(pl.cdiv(v, col_block),),
        in_specs=[
            pl.BlockSpec((d, col_block), lambda j: (0, j)),
            pl.BlockSpec((d, col_block), lambda j: (0, j)),
        ],
        out_specs=pl.BlockSpec((col_block, ROW), lambda j: (j, 0)),
        out_shape=jax.ShapeDtypeStruct((v, ROW), jnp.float32),
    )(base_t, super_t)


def _out_body(in_ref, out_ref):
    out_ref[...] = jnp.transpose(in_ref[:, :, 0:DIM], (0, 2, 1))


def _tc_out_transpose(a, l_block=4):
    l, b, _ = a.shape
    return pl.pallas_call(
        _out_body,
        grid=(l // l_block,),
        in_specs=[pl.BlockSpec((l_block, b, ROW), lambda i: (i, 0, 0))],
        out_specs=pl.BlockSpec((l_block, DIM, b), lambda i: (i, 0, 0)),
        out_shape=jax.ShapeDtypeStruct((l, DIM, b), jnp.float32),
    )(a)


@functools.cache
def _build_sc_kernel(n_tokens: int):
    info = plsc.get_sparse_core_info()
    n_workers = info.num_cores * info.num_subcores
    per_worker = n_tokens // n_workers
    n_chunks = per_worker // CHUNK
    n_iters = n_chunks // NSLOT
    assert per_worker * n_workers == n_tokens
    assert n_iters * NSLOT == n_chunks

    mesh = plsc.VectorSubcoreMesh(core_axis_name="c", subcore_axis_name="s")

    @functools.partial(
        pl.kernel,
        mesh=mesh,
        out_type=jax.ShapeDtypeStruct((n_tokens, ROW), jnp.float32),
        compiler_params=pltpu.CompilerParams(use_tc_tiling_on_sc=False),
        scratch_types=[
            pltpu.VMEM((per_worker,), jnp.int32),
            pltpu.VMEM((per_worker,), jnp.float32),
            pltpu.VMEM((NSLOT, CHUNK, ROW), jnp.float32),
            pltpu.VMEM((NSLOT, CHUNK, ROW), jnp.float32),
            pltpu.SemaphoreType.DMA((NSLOT,)),
            pltpu.SemaphoreType.DMA((NSLOT,)),
        ],
    )
    def sc_combine(x_hbm, ctx_hbm, tab_hbm, out_hbm,
                   idx_all, ctx_all, g_v, o_v, gsem, osem):
        wid = lax.axis_index("s") * info.num_cores + lax.axis_index("c")
        w_base = wid * per_worker

        pltpu.sync_copy(x_hbm.at[pl.ds(w_base, per_worker)], idx_all)
        pltpu.sync_copy(ctx_hbm.at[pl.ds(w_base, per_worker)], ctx_all)

        def fire_gather(c, k):
            idx_slice = idx_all.at[pl.ds(c * CHUNK, CHUNK)]
            pltpu.async_copy(tab_hbm.at[idx_slice], g_v.at[k], gsem.at[k])

        def drain_gather(c, k):
            idx_slice = idx_all.at[pl.ds(c * CHUNK, CHUNK)]
            pltpu.make_async_copy(
                tab_hbm.at[idx_slice], g_v.at[k], gsem.at[k]).wait()

        def fire_scatter(c, k):
            pltpu.async_copy(
                o_v.at[k], out_hbm.at[pl.ds(w_base + c * CHUNK, CHUNK)],
                osem.at[k])

        def drain_scatter(c, k):
            pltpu.make_async_copy(
                o_v.at[k], out_hbm.at[pl.ds(w_base + c * CHUNK, CHUNK)],
                osem.at[k]).wait()

        def compute(c, k):
            g_ref = g_v.at[k]
            o_ref = o_v.at[k]
            goff = c * CHUNK

            def group(tg, carry):
                t0 = tg * LANES
                cv16 = ctx_all[pl.ds(goff + t0, LANES)]
                for j in range(LANES):
                    cb = _bcast_lane(cv16, j)
                    t = t0 + j
                    for d in range(DIM // LANES):
                        sl = pl.ds(d * LANES, LANES)
                        sh = pl.ds(DIM + d * LANES, LANES)
                        o_ref[t, sl] = g_ref[t, sl] + cb * g_ref[t, sh]
                return carry

            lax.fori_loop(0, CHUNK // LANES, group, 0)

        for k in range(NSLOT - 1):
            fire_gather(k, k)

        def iter_body(q, carry):
            c0 = q * NSLOT
            for k in range(NSLOT):
                c = c0 + k
                drain_gather(c, k)
                @pl.when(q > 0)
                def _():
                    drain_scatter(c - NSLOT, k)
                compute(c, k)
                fire_scatter(c, k)
                kn = (k + NSLOT - 1) % NSLOT
                if k == 0:
                    fire_gather(c + NSLOT - 1, kn)
                else:
                    @pl.when(q < n_iters - 1)
                    def _():
                        fire_gather(c + NSLOT - 1, kn)
            return carry

        lax.fori_loop(0, n_iters, iter_body, 0)
        for k in range(NSLOT):
            drain_scatter(n_chunks - NSLOT + k, (n_chunks - NSLOT + k) % NSLOT)

    return sc_combine


def kernel(x, context_vector, base_table, super_table):
    b, l = x.shape
    n_tokens = b * l
    tab = _tc_interleave_tables(base_table.T, super_table.T)
    xt = jnp.swapaxes(x, 0, 1).reshape(n_tokens).astype(jnp.int32)
    ct = jnp.swapaxes(context_vector, 0, 1).reshape(n_tokens)
    sc = _build_sc_kernel(n_tokens)
    out = sc(xt, ct, tab)
    out_t = _tc_out_transpose(out.reshape(l, b, ROW))
    return jnp.transpose(out_t, (2, 0, 1))


# speedup vs baseline: 5.4903x; 1.0074x over previous
import functools

import jax
import jax.numpy as jnp
from jax import lax
from jax.experimental import pallas as pl
from jax.experimental.pallas import tpu as pltpu
from jax.experimental.pallas import tpu_sc as plsc

DIM = 64
ROW = 128
LANES = 16
CHUNK = 64
NSLOT = 4


def _bcast_lane(v, j):
    return lax.gather(
        v,
        jnp.full((LANES, 1), j, jnp.int32),
        lax.GatherDimensionNumbers(
            offset_dims=(), collapsed_slice_dims=(0,), start_index_map=(0,)),
        (1,),
        mode=lax.GatherScatterMode.PROMISE_IN_BOUNDS)


def _interleave_body(b_ref, s_ref, out_ref):
    out_ref[...] = jnp.concatenate([b_ref[...], s_ref[...]], axis=0).T


def _tc_interleave_tables(base_t, super_t, col_block=8192):
    d, v = base_t.shape
    return pl.pallas_call(
        _interleave_body,
        grid=(pl.cdiv(v, col_block),),
        in_specs=[
            pl.BlockSpec((d, col_block), lambda j: (0, j)),
            pl.BlockSpec((d, col_block), lambda j: (0, j)),
        ],
        out_specs=pl.BlockSpec((col_block, ROW), lambda j: (j, 0)),
        out_shape=jax.ShapeDtypeStruct((v, ROW), jnp.float32),
    )(base_t, super_t)


def _out_body(in_ref, out_ref):
    out_ref[...] = jnp.transpose(in_ref[:, :, 0:DIM], (0, 2, 1))


def _tc_out_transpose(a, l_total, l_block=4):
    lh, b, _ = a.shape
    return pl.pallas_call(
        _out_body,
        grid=(lh // l_block,),
        in_specs=[pl.BlockSpec((l_block, b, ROW), lambda i: (i, 0, 0))],
        out_specs=pl.BlockSpec((l_block, DIM, b), lambda i: (i, 0, 0)),
        out_shape=jax.ShapeDtypeStruct((l_total, DIM, b), jnp.float32),
    )(a)


def _out_body2(dst_ref, in_ref, out_ref):
    out_ref[...] = jnp.transpose(in_ref[:, :, 0:DIM], (0, 2, 1))


def _tc_out_transpose_into(dst, a, l_off, l_block=4):
    lh, b, _ = a.shape
    blk_off = l_off // l_block
    return pl.pallas_call(
        _out_body2,
        grid=(lh // l_block,),
        in_specs=[
            pl.BlockSpec(memory_space=pl.ANY),
            pl.BlockSpec((l_block, b, ROW), lambda i: (i, 0, 0)),
        ],
        out_specs=pl.BlockSpec(
            (l_block, DIM, b), lambda i: (i + blk_off, 0, 0)),
        out_shape=jax.ShapeDtypeStruct(dst.shape, jnp.float32),
        input_output_aliases={0: 0},
    )(dst, a)


@functools.cache
def _build_sc_kernel(n_tokens: int):
    info = plsc.get_sparse_core_info()
    n_workers = info.num_cores * info.num_subcores
    per_worker = n_tokens // n_workers
    n_chunks = per_worker // CHUNK
    n_iters = n_chunks // NSLOT
    assert per_worker * n_workers == n_tokens
    assert n_iters * NSLOT == n_chunks

    mesh = plsc.VectorSubcoreMesh(core_axis_name="c", subcore_axis_name="s")

    @functools.partial(
        pl.kernel,
        mesh=mesh,
        out_type=jax.ShapeDtypeStruct((n_tokens, ROW), jnp.float32),
        compiler_params=pltpu.CompilerParams(use_tc_tiling_on_sc=False),
        scratch_types=[
            pltpu.VMEM((per_worker,), jnp.int32),
            pltpu.VMEM((per_worker,), jnp.float32),
            pltpu.VMEM((NSLOT, CHUNK, ROW), jnp.float32),
            pltpu.VMEM((NSLOT, CHUNK, ROW), jnp.float32),
            pltpu.SemaphoreType.DMA((NSLOT,)),
            pltpu.SemaphoreType.DMA((NSLOT,)),
        ],
    )
    def sc_combine(x_hbm, ctx_hbm, tab_hbm, out_hbm,
                   idx_all, ctx_all, g_v, o_v, gsem, osem):
        wid = lax.axis_index("s") * info.num_cores + lax.axis_index("c")
        w_base = wid * per_worker

        pltpu.sync_copy(x_hbm.at[pl.ds(w_base, per_worker)], idx_all)
        pltpu.sync_copy(ctx_hbm.at[pl.ds(w_base, per_worker)], ctx_all)

        def fire_gather(c, k):
            idx_slice = idx_all.at[pl.ds(c * CHUNK, CHUNK)]
            pltpu.async_copy(tab_hbm.at[idx_slice], g_v.at[k], gsem.at[k])

        def drain_gather(c, k):
            idx_slice = idx_all.at[pl.ds(c * CHUNK, CHUNK)]
            pltpu.make_async_copy(
                tab_hbm.at[idx_slice], g_v.at[k], gsem.at[k]).wait()

        def fire_scatter(c, k):
            pltpu.async_copy(
                o_v.at[k], out_hbm.at[pl.ds(w_base + c * CHUNK, CHUNK)],
                osem.at[k])

        def drain_scatter(c, k):
            pltpu.make_async_copy(
                o_v.at[k], out_hbm.at[pl.ds(w_base + c * CHUNK, CHUNK)],
                osem.at[k]).wait()

        def compute(c, k):
            g_ref = g_v.at[k]
            o_ref = o_v.at[k]
            goff = c * CHUNK

            def group(tg, carry):
                t0 = tg * LANES
                cv16 = ctx_all[pl.ds(goff + t0, LANES)]
                for j in range(LANES):
                    cb = _bcast_lane(cv16, j)
                    t = t0 + j
                    for d in range(DIM // LANES):
                        sl = pl.ds(d * LANES, LANES)
                        sh = pl.ds(DIM + d * LANES, LANES)
                        o_ref[t, sl] = g_ref[t, sl] + cb * g_ref[t, sh]
                return carry

            lax.fori_loop(0, CHUNK // LANES, group, 0)

        for k in range(NSLOT - 1):
            fire_gather(k, k)

        def iter_body(q, carry):
            c0 = q * NSLOT
            for k in range(NSLOT):
                c = c0 + k
                drain_gather(c, k)
                @pl.when(q > 0)
                def _():
                    drain_scatter(c - NSLOT, k)
                compute(c, k)
                fire_scatter(c, k)
                kn = (k + NSLOT - 1) % NSLOT
                if k == 0:
                    fire_gather(c + NSLOT - 1, kn)
                else:
                    @pl.when(q < n_iters - 1)
                    def _():
                        fire_gather(c + NSLOT - 1, kn)
            return carry

        lax.fori_loop(0, n_iters, iter_body, 0)
        for k in range(NSLOT):
            drain_scatter(n_chunks - NSLOT + k, (n_chunks - NSLOT + k) % NSLOT)

    return sc_combine


def kernel(x, context_vector, base_table, super_table):
    b, l = x.shape
    n_tokens = b * l
    tab = _tc_interleave_tables(base_table.T, super_table.T)
    xt = jnp.swapaxes(x, 0, 1).reshape(n_tokens).astype(jnp.int32)
    ct = jnp.swapaxes(context_vector, 0, 1).reshape(n_tokens)
    n_half = n_tokens // 2
    l_half = l // 2
    sc = _build_sc_kernel(n_half)
    out1 = sc(xt[:n_half], ct[:n_half], tab)
    out2 = sc(xt[n_half:], ct[n_half:], tab)
    o_part = _tc_out_transpose(out1.reshape(l_half, b, ROW), l)
    o_full = _tc_out_transpose_into(o_part, out2.reshape(l_half, b, ROW),
                                    l_half)
    return jnp.transpose(o_full, (2, 0, 1))
